# Initial kernel scaffold; baseline (speedup 1.0000x reference)
import jax
import jax.numpy as jnp
from jax.experimental import pallas as pl


def kernel(x, edge_index, batch, W0, b0, W1, b1, W2, b2):
    raise NotImplementedError("write your pallas kernel here")


# speedup vs baseline: 9.9166x; 9.9166x over previous
import functools

import jax
import jax.numpy as jnp
from jax import lax
from jax.experimental import pallas as pl
from jax.experimental.pallas import tpu as pltpu
from jax.experimental.pallas import tpu_sc as plsc

N = 10000
NP = 10240
E = 320000
D = 128
G = 128
NC = 2
NS = 16
NW = NC * NS
CHUNK = 128
CH = 79
TPT = CH * CHUNK
EP = NW * TPT
SLAB = NP // NS
RB = 10
BR = NP // RB


def _sc_mesh():
    return plsc.VectorSubcoreMesh(core_axis_name="c", subcore_axis_name="s")


def _sc_degree(dst_idx):
    @functools.partial(
        pl.kernel,
        out_type=jax.ShapeDtypeStruct((NC, NP, 16), jnp.float32),
        mesh=_sc_mesh(),
        scratch_types=[
            pltpu.VMEM((CH, CHUNK), jnp.int32),
            pltpu.VMEM((CHUNK, 16), jnp.float32),
            pltpu.VMEM((CHUNK, 16), jnp.float32),
            pltpu.VMEM_SHARED((NP, 16), jnp.float32),
        ],
    )
    def deg_kernel(dst_hbm, out_hbm, dst_v, ones_v, zero_v, acc):
        cid = lax.axis_index("c")
        sid = lax.axis_index("s")
        wid = cid * NS + sid

        def fill(i, carry):
            ones_v[i, :] = jnp.ones((16,), jnp.float32)
            zero_v[i, :] = jnp.zeros((16,), jnp.float32)
            return carry

        lax.fori_loop(0, CHUNK, fill, 0)
        for r in range(SLAB // CHUNK):
            pltpu.sync_copy(zero_v, acc.at[pl.ds(sid * SLAB + r * CHUNK, CHUNK)])
        pltpu.sync_copy(dst_hbm.at[wid], dst_v)
        plsc.subcore_barrier()

        def step(j, carry):
            pltpu.sync_copy(ones_v, acc.at[dst_v.at[j]], add=True)
            return carry

        lax.fori_loop(0, CH, step, 0)
        plsc.subcore_barrier()
        pltpu.sync_copy(acc.at[pl.ds(sid * SLAB, SLAB)],
                        out_hbm.at[cid, pl.ds(sid * SLAB, SLAB)])

    return deg_kernel(dst_idx)


def _sc_spmm(u, src_idx, dst_idx):
    @functools.partial(
        pl.kernel,
        out_type=jax.ShapeDtypeStruct((NC, NP, D), jnp.float32),
        mesh=_sc_mesh(),
        scratch_types=[
            pltpu.VMEM((CH, CHUNK), jnp.int32),
            pltpu.VMEM((CH, CHUNK), jnp.int32),
            pltpu.VMEM((CHUNK, D), jnp.float32),
            pltpu.VMEM_SHARED((NP, D), jnp.float32),
            pltpu.SemaphoreType.DMA,
        ],
    )
    def spmm_kernel(u_hbm, src_hbm, dst_hbm, out_hbm,
                    src_v, dst_v, rows_v, acc, sem):
        cid = lax.axis_index("c")
        sid = lax.axis_index("s")
        wid = cid * NS + sid

        def fill(i, carry):
            for k in range(D // 16):
                rows_v[i, pl.ds(k * 16, 16)] = jnp.zeros((16,), jnp.float32)
            return carry

        lax.fori_loop(0, CHUNK, fill, 0)
        for r in range(SLAB // CHUNK):
            pltpu.sync_copy(rows_v, acc.at[pl.ds(sid * SLAB + r * CHUNK, CHUNK)])
        pltpu.sync_copy(src_hbm.at[wid], src_v)
        pltpu.sync_copy(dst_hbm.at[wid], dst_v)
        plsc.subcore_barrier()

        def step(j, carry):
            pltpu.async_copy(u_hbm.at[src_v.at[j]], rows_v, sem).wait()
            pltpu.sync_copy(rows_v, acc.at[dst_v.at[j]], add=True)
            return carry

        lax.fori_loop(0, CH, step, 0)
        plsc.subcore_barrier()
        pltpu.sync_copy(acc.at[pl.ds(sid * SLAB, SLAB)],
                        out_hbm.at[cid, pl.ds(sid * SLAB, SLAB)])

    return spmm_kernel(u, src_idx, dst_idx)


def _tcA_body(deg_ref, x_ref, w_ref, dinv_ref, u_ref):
    i = pl.program_id(0)
    d = deg_ref[0, :, 0:1] + deg_ref[1, :, 0:1] + 1.0
    rowid = i * BR + lax.broadcasted_iota(jnp.int32, (BR, 1), 0)
    dinv = jnp.where(rowid < N, lax.rsqrt(d), 0.0)
    dinv_ref[...] = dinv
    u_ref[...] = dinv * jnp.dot(x_ref[...], w_ref[...],
                                preferred_element_type=jnp.float32)


def _tc_prelayer(deg2, x_pad, W0):
    return pl.pallas_call(
        _tcA_body,
        grid=(RB,),
        in_specs=[
            pl.BlockSpec((NC, BR, 16), lambda i: (0, i, 0)),
            pl.BlockSpec((BR, D), lambda i: (i, 0)),
            pl.BlockSpec((D, D), lambda i: (0, 0)),
        ],
        out_specs=[
            pl.BlockSpec((BR, 1), lambda i: (i, 0)),
            pl.BlockSpec((BR, D), lambda i: (i, 0)),
        ],
        out_shape=[
            jax.ShapeDtypeStruct((NP, 1), jnp.float32),
            jax.ShapeDtypeStruct((NP, D), jnp.float32),
        ],
    )(deg2, x_pad, W0)


def _post_mid_body(s2_ref, u_ref, dinv_ref, b_ref, w_ref, out_ref, un_ref):
    s = s2_ref[0] + s2_ref[1]
    o = jnp.maximum(dinv_ref[...] * (s + u_ref[...]) + b_ref[...][None, :], 0.0)
    out_ref[...] = o
    un_ref[...] = dinv_ref[...] * jnp.dot(o, w_ref[...],
                                          preferred_element_type=jnp.float32)


def _tc_post_mid(s2, u, dinv, b, Wn):
    return pl.pallas_call(
        _post_mid_body,
        grid=(RB,),
        in_specs=[
            pl.BlockSpec((NC, BR, D), lambda i: (0, i, 0)),
            pl.BlockSpec((BR, D), lambda i: (i, 0)),
            pl.BlockSpec((BR, 1), lambda i: (i, 0)),
            pl.BlockSpec((D,), lambda i: (0,)),
            pl.BlockSpec((D, D), lambda i: (0, 0)),
        ],
        out_specs=[
            pl.BlockSpec((BR, D), lambda i: (i, 0)),
            pl.BlockSpec((BR, D), lambda i: (i, 0)),
        ],
        out_shape=[
            jax.ShapeDtypeStruct((NP, D), jnp.float32),
            jax.ShapeDtypeStruct((NP, D), jnp.float32),
        ],
    )(s2, u, dinv, b, Wn)


def _post_last_body(s2_ref, u_ref, dinv_ref, b_ref, out_ref):
    s = s2_ref[0] + s2_ref[1]
    out_ref[...] = jnp.maximum(dinv_ref[...] * (s + u_ref[...]) + b_ref[...][None, :], 0.0)


def _tc_post_last(s2, u, dinv, b):
    return pl.pallas_call(
        _post_last_body,
        grid=(RB,),
        in_specs=[
            pl.BlockSpec((NC, BR, D), lambda i: (0, i, 0)),
            pl.BlockSpec((BR, D), lambda i: (i, 0)),
            pl.BlockSpec((BR, 1), lambda i: (i, 0)),
            pl.BlockSpec((D,), lambda i: (0,)),
        ],
        out_specs=pl.BlockSpec((BR, D), lambda i: (i, 0)),
        out_shape=jax.ShapeDtypeStruct((NP, D), jnp.float32),
    )(s2, u, dinv, b)


def _pool_body(b3_ref, h_ref, pool_ref, acc, cnt):
    i = pl.program_id(0)
    m = (lax.broadcasted_iota(jnp.int32, (G, BR), 0) == b3_ref[0]).astype(jnp.float32)
    part = jnp.dot(m, h_ref[...], preferred_element_type=jnp.float32)
    c = jnp.sum(m, axis=1, keepdims=True)

    @pl.when(i == 0)
    def _():
        acc[...] = part
        cnt[...] = c

    @pl.when(i > 0)
    def _():
        acc[...] += part
        cnt[...] += c

    @pl.when(i == RB - 1)
    def _():
        pool_ref[...] = acc[...] / jnp.maximum(cnt[...], 1.0)


def _tc_pool(batch3, h):
    return pl.pallas_call(
        _pool_body,
        grid=(RB,),
        in_specs=[
            pl.BlockSpec((1, 1, BR), lambda i: (i, 0, 0)),
            pl.BlockSpec((BR, 3 * D), lambda i: (i, 0)),
        ],
        out_specs=pl.BlockSpec((G, 3 * D), lambda i: (0, 0)),
        out_shape=jax.ShapeDtypeStruct((G, 3 * D), jnp.float32),
        scratch_shapes=[
            pltpu.VMEM((G, 3 * D), jnp.float32),
            pltpu.VMEM((G, 1), jnp.float32),
        ],
    )(batch3, h)


def kernel(x, edge_index, batch, W0, b0, W1, b1, W2, b2):
    src = edge_index[0].astype(jnp.int32)
    dst = edge_index[1].astype(jnp.int32)
    pad = jnp.full((EP - E,), N, jnp.int32)
    src_idx = jnp.concatenate([src, pad]).reshape(NW, CH, CHUNK)
    dst_idx = jnp.concatenate([dst, pad]).reshape(NW, CH, CHUNK)
    x_pad = jnp.pad(x, ((0, NP - N), (0, 0)))
    batch3 = jnp.concatenate(
        [batch.astype(jnp.int32), jnp.full((NP - N,), G, jnp.int32)]
    ).reshape(RB, 1, BR)

    deg2 = _sc_degree(dst_idx)
    dinv, u = _tc_prelayer(deg2, x_pad, W0)

    outs = []
    for Wn, b in ((W1, b0), (W2, b1), (None, b2)):
        s2 = _sc_spmm(u, src_idx, dst_idx)
        if Wn is None:
            o = _tc_post_last(s2, u, dinv, b)
        else:
            o, u = _tc_post_mid(s2, u, dinv, b, Wn)
        outs.append(o)

    h = jnp.concatenate(outs, axis=1)
    pool = _tc_pool(batch3, h)
    return (pool, h[:N])
